# baseline (device time: 118228 ns/iter reference)
import numpy as np
import jax
import jax.numpy as jnp
from jax import lax
from jax.experimental import pallas as pl
from jax.experimental.pallas import tpu as pltpu

N_DEV = 32
CW_HOPS = N_DEV // 2
CCW_HOPS = N_DEV - 1 - CW_HOPS

_PLANE = [(0, 0), (1, 0), (1, 1), (0, 1), (0, 2), (1, 2), (1, 3), (0, 3)]


def _logical(x, y, z):
    return 8 * z + _PLANE.index((x, y))


_FACE = [(0, 0), (1, 0), (2, 0), (3, 0), (3, 1), (2, 1), (1, 1), (0, 1),
         (0, 2), (1, 2), (2, 2), (3, 2), (3, 3), (2, 3), (1, 3), (0, 3)]
_CYCLE = [(0, y, z) for (y, z) in _FACE] + [(1, y, z) for (y, z) in reversed(_FACE)]
_HAM = [_logical(*c) for c in _CYCLE]
_POS = {l: p for p, l in enumerate(_HAM)}

TAB_L = np.array([[_HAM[(_POS[l] - h - 1) % N_DEV] for h in range(CW_HOPS)]
                  for l in range(N_DEV)], np.int32)
TAB_R = np.array([[_HAM[(_POS[l] + h + 1) % N_DEV] for h in range(CW_HOPS)]
                  for l in range(N_DEV)], np.int32)


def kernel(x, w_mat, scale_x, scale_w):
    m_per, k = x.shape
    n_per = w_mat.shape[1]

    my = lax.axis_index("i")
    lefts = jnp.asarray(TAB_L)[my]
    rights = jnp.asarray(TAB_R)[my]
    my_arr = jnp.full((1,), my, jnp.int32)

    def body(x_ref, w_ref, sx_ref, sw_ref, my_ref, lefts_ref, rights_ref,
             out_ref, cw_ref, ccw_ref, res_send_buf, res_recv_buf,
             cw_send, cw_recv, ccw_send, ccw_recv,
             res_send_sems, res_recv_sems):
        me = my_ref[0]
        left = lefts_ref[0]
        right = rights_ref[0]

        barrier_sem = pltpu.get_barrier_semaphore()
        pl.semaphore_signal(barrier_sem, inc=1, device_id=(left,),
                            device_id_type=pl.DeviceIdType.MESH)
        pl.semaphore_signal(barrier_sem, inc=1, device_id=(right,),
                            device_id_type=pl.DeviceIdType.MESH)
        pl.semaphore_wait(barrier_sem, 2)

        scale = sx_ref[0] * sw_ref[0]
        xv = x_ref[...]

        def result(w_chunk):
            acc = jnp.dot(xv, w_chunk, preferred_element_type=jnp.int32)
            y = acc.astype(jnp.float32) * scale
            return y * jax.nn.sigmoid(y)

        half = k // 4

        def ring_rdma(buf, sends, recvs, h, s, dev):
            return pltpu.make_async_remote_copy(
                src_ref=buf.at[h, pl.ds(s * half, half), :],
                dst_ref=buf.at[h + 1, pl.ds(s * half, half), :],
                send_sem=sends.at[h, s], recv_sem=recvs.at[h, s],
                device_id=(dev,), device_id_type=pl.DeviceIdType.MESH,
            )

        def res_rdma(side, h, dev):
            return pltpu.make_async_remote_copy(
                src_ref=res_send_buf.at[side, h],
                dst_ref=res_recv_buf.at[side, h],
                send_sem=res_send_sems.at[side, h],
                recv_sem=res_recv_sems.at[side, h],
                device_id=(dev,), device_id_type=pl.DeviceIdType.MESH,
            )

        cw = [[ring_rdma(cw_ref, cw_send, cw_recv, h, s, right) for s in range(4)]
              for h in range(CW_HOPS)]
        ccw = [[ring_rdma(ccw_ref, ccw_send, ccw_recv, h, s, left) for s in range(4)]
               for h in range(CCW_HOPS)]

        cw_ref[0] = w_ref[...]
        ccw_ref[0] = w_ref[...]
        for s in range(4):
            cw[0][s].start()
            ccw[0][s].start()

        out_ref[pl.ds(me * m_per, m_per), :] = result(w_ref[...])

        res_sends = []
        for h in range(CW_HOPS):
            for s in range(4):
                cw[h][s].wait_recv()
                if h + 1 < CW_HOPS:
                    cw[h + 1][s].start()
                if h < CCW_HOPS:
                    ccw[h][s].wait_recv()
                    if h + 1 < CCW_HOPS:
                        ccw[h + 1][s].start()
            res_send_buf[0, h] = result(cw_ref[h + 1]).astype(jnp.bfloat16)
            r = res_rdma(0, h, lefts_ref[h])
            r.start()
            res_sends.append(r)
            if h < CCW_HOPS:
                res_send_buf[1, h] = result(ccw_ref[h + 1]).astype(jnp.bfloat16)
                r = res_rdma(1, h, rights_ref[h])
                r.start()
                res_sends.append(r)

        for h in range(CW_HOPS):
            res_rdma(0, h, right).wait_recv()
            out_ref[pl.ds(rights_ref[h] * m_per, m_per), :] = (
                res_recv_buf[0, h].astype(jnp.float32))
        for h in range(CCW_HOPS):
            res_rdma(1, h, left).wait_recv()
            out_ref[pl.ds(lefts_ref[h] * m_per, m_per), :] = (
                res_recv_buf[1, h].astype(jnp.float32))

        for pair in cw + ccw:
            for r in pair:
                r.wait_send()
        for r in res_sends:
            r.wait_send()

    out_shape = jax.ShapeDtypeStruct((N_DEV * m_per, n_per), jnp.float32)
    return pl.pallas_call(
        body,
        out_shape=out_shape,
        in_specs=[
            pl.BlockSpec(memory_space=pltpu.VMEM),
            pl.BlockSpec(memory_space=pltpu.VMEM),
            pl.BlockSpec(memory_space=pltpu.SMEM),
            pl.BlockSpec(memory_space=pltpu.SMEM),
            pl.BlockSpec(memory_space=pltpu.SMEM),
            pl.BlockSpec(memory_space=pltpu.SMEM),
            pl.BlockSpec(memory_space=pltpu.SMEM),
        ],
        out_specs=pl.BlockSpec(memory_space=pltpu.VMEM),
        scratch_shapes=[
            pltpu.VMEM((CW_HOPS + 1, k, n_per), w_mat.dtype),
            pltpu.VMEM((CCW_HOPS + 1, k, n_per), w_mat.dtype),
            pltpu.VMEM((2, CW_HOPS, m_per, n_per), jnp.bfloat16),
            pltpu.VMEM((2, CW_HOPS, m_per, n_per), jnp.bfloat16),
            pltpu.SemaphoreType.DMA((CW_HOPS, 4)),
            pltpu.SemaphoreType.DMA((CW_HOPS, 4)),
            pltpu.SemaphoreType.DMA((CCW_HOPS, 4)),
            pltpu.SemaphoreType.DMA((CCW_HOPS, 4)),
            pltpu.SemaphoreType.DMA((2, CW_HOPS)),
            pltpu.SemaphoreType.DMA((2, CW_HOPS)),
        ],
        compiler_params=pltpu.CompilerParams(collective_id=0),
    )(x, w_mat, scale_x, scale_w, my_arr, lefts, rights)


# device time: 116995 ns/iter; 1.0105x vs baseline; 1.0105x over previous
import numpy as np
import jax
import jax.numpy as jnp
from jax import lax
from jax.experimental import pallas as pl
from jax.experimental.pallas import tpu as pltpu

N_DEV = 32
CW_HOPS = N_DEV // 2
CCW_HOPS = N_DEV - 1 - CW_HOPS

_PLANE = [(0, 0), (1, 0), (1, 1), (0, 1), (0, 2), (1, 2), (1, 3), (0, 3)]


def _logical(x, y, z):
    return 8 * z + _PLANE.index((x, y))


_FACE = [(0, 0), (1, 0), (2, 0), (3, 0), (3, 1), (2, 1), (1, 1), (0, 1),
         (0, 2), (1, 2), (2, 2), (3, 2), (3, 3), (2, 3), (1, 3), (0, 3)]
_CYCLE = [(0, y, z) for (y, z) in _FACE] + [(1, y, z) for (y, z) in reversed(_FACE)]
_HAM = [_logical(*c) for c in _CYCLE]
_POS = {l: p for p, l in enumerate(_HAM)}

TAB_L = np.array([[_HAM[(_POS[l] - h - 1) % N_DEV] for h in range(CW_HOPS)]
                  for l in range(N_DEV)], np.int32)
TAB_R = np.array([[_HAM[(_POS[l] + h + 1) % N_DEV] for h in range(CW_HOPS)]
                  for l in range(N_DEV)], np.int32)


def kernel(x, w_mat, scale_x, scale_w):
    m_per, k = x.shape
    n_per = w_mat.shape[1]

    my = lax.axis_index("i")
    lefts = jnp.asarray(TAB_L)[my]
    rights = jnp.asarray(TAB_R)[my]
    my_arr = jnp.full((1,), my, jnp.int32)

    def body(x_ref, w_ref, sx_ref, sw_ref, my_ref, lefts_ref, rights_ref,
             out_ref, cw_ref, ccw_ref, res_send_buf, res_recv_buf,
             cw_send, cw_recv, ccw_send, ccw_recv,
             res_send_sems, res_recv_sems):
        me = my_ref[0]
        left = lefts_ref[0]
        right = rights_ref[0]

        barrier_sem = pltpu.get_barrier_semaphore()
        pl.semaphore_signal(barrier_sem, inc=1, device_id=(left,),
                            device_id_type=pl.DeviceIdType.MESH)
        pl.semaphore_signal(barrier_sem, inc=1, device_id=(right,),
                            device_id_type=pl.DeviceIdType.MESH)
        pl.semaphore_wait(barrier_sem, 2)

        scale = sx_ref[0] * sw_ref[0]
        xv = x_ref[...]

        def result(w_chunk):
            acc = jnp.dot(xv, w_chunk, preferred_element_type=jnp.int32)
            y = acc.astype(jnp.float32) * scale
            return y * jax.nn.sigmoid(y)

        half = k // 2

        def ring_rdma(buf, sends, recvs, h, s, dev):
            return pltpu.make_async_remote_copy(
                src_ref=buf.at[h, pl.ds(s * half, half), :],
                dst_ref=buf.at[h + 1, pl.ds(s * half, half), :],
                send_sem=sends.at[h, s], recv_sem=recvs.at[h, s],
                device_id=(dev,), device_id_type=pl.DeviceIdType.MESH,
            )

        def res_rdma(side, h, dev):
            return pltpu.make_async_remote_copy(
                src_ref=res_send_buf.at[side, h],
                dst_ref=res_recv_buf.at[side, h],
                send_sem=res_send_sems.at[side, h],
                recv_sem=res_recv_sems.at[side, h],
                device_id=(dev,), device_id_type=pl.DeviceIdType.MESH,
            )

        cw = [[ring_rdma(cw_ref, cw_send, cw_recv, h, s, right) for s in range(2)]
              for h in range(CW_HOPS)]
        ccw = [[ring_rdma(ccw_ref, ccw_send, ccw_recv, h, s, left) for s in range(2)]
               for h in range(CCW_HOPS)]

        cw_ref[0] = w_ref[...]
        ccw_ref[0] = w_ref[...]
        for s in range(2):
            cw[0][s].start()
            ccw[0][s].start()

        out_ref[pl.ds(me * m_per, m_per), :] = result(w_ref[...])

        res_sends = []
        for h in range(CW_HOPS):
            for s in range(2):
                cw[h][s].wait_recv()
                if h + 1 < CW_HOPS:
                    cw[h + 1][s].start()
                if h < CCW_HOPS:
                    ccw[h][s].wait_recv()
                    if h + 1 < CCW_HOPS:
                        ccw[h + 1][s].start()
            res_send_buf[0, h] = result(cw_ref[h + 1]).astype(jnp.bfloat16)
            r = res_rdma(0, h, lefts_ref[h])
            r.start()
            res_sends.append(r)
            if h < CCW_HOPS:
                res_send_buf[1, h] = result(ccw_ref[h + 1]).astype(jnp.bfloat16)
                r = res_rdma(1, h, rights_ref[h])
                r.start()
                res_sends.append(r)

        for h in range(CW_HOPS):
            res_rdma(0, h, right).wait_recv()
            out_ref[pl.ds(rights_ref[h] * m_per, m_per), :] = (
                res_recv_buf[0, h].astype(jnp.float32))
        for h in range(CCW_HOPS):
            res_rdma(1, h, left).wait_recv()
            out_ref[pl.ds(lefts_ref[h] * m_per, m_per), :] = (
                res_recv_buf[1, h].astype(jnp.float32))

        for pair in cw + ccw:
            for r in pair:
                r.wait_send()
        for r in res_sends:
            r.wait_send()

    out_shape = jax.ShapeDtypeStruct((N_DEV * m_per, n_per), jnp.float32)
    return pl.pallas_call(
        body,
        out_shape=out_shape,
        in_specs=[
            pl.BlockSpec(memory_space=pltpu.VMEM),
            pl.BlockSpec(memory_space=pltpu.VMEM),
            pl.BlockSpec(memory_space=pltpu.SMEM),
            pl.BlockSpec(memory_space=pltpu.SMEM),
            pl.BlockSpec(memory_space=pltpu.SMEM),
            pl.BlockSpec(memory_space=pltpu.SMEM),
            pl.BlockSpec(memory_space=pltpu.SMEM),
        ],
        out_specs=pl.BlockSpec(memory_space=pltpu.VMEM),
        scratch_shapes=[
            pltpu.VMEM((CW_HOPS + 1, k, n_per), w_mat.dtype),
            pltpu.VMEM((CCW_HOPS + 1, k, n_per), w_mat.dtype),
            pltpu.VMEM((2, CW_HOPS, m_per, n_per), jnp.bfloat16),
            pltpu.VMEM((2, CW_HOPS, m_per, n_per), jnp.bfloat16),
            pltpu.SemaphoreType.DMA((CW_HOPS, 2)),
            pltpu.SemaphoreType.DMA((CW_HOPS, 2)),
            pltpu.SemaphoreType.DMA((CCW_HOPS, 2)),
            pltpu.SemaphoreType.DMA((CCW_HOPS, 2)),
            pltpu.SemaphoreType.DMA((2, CW_HOPS)),
            pltpu.SemaphoreType.DMA((2, CW_HOPS)),
        ],
        compiler_params=pltpu.CompilerParams(collective_id=0),
    )(x, w_mat, scale_x, scale_w, my_arr, lefts, rights)
